# Initial kernel scaffold; baseline (speedup 1.0000x reference)
import jax
import jax.numpy as jnp
from jax.experimental import pallas as pl


def kernel(x, W1, b1, W2, b2):
    raise NotImplementedError("write your pallas kernel here")


# speedup vs baseline: 2.5739x; 2.5739x over previous
import functools

import jax
import jax.numpy as jnp
from jax import lax
from jax.experimental import pallas as pl
from jax.experimental.pallas import tpu as pltpu
from jax.experimental.pallas import tpu_sc as plsc

B, T, C, H, W = 8, 8, 96, 64, 64
N = H * W
K = N // 2
D = T * C
NB = 512
CHUNK = 64
WPB = 4
ROWS_PER_W = K // WPB
EPS = 1e-6


def _mt_body(x_ref, xt_ref, mean_ref):
    xb = x_ref[0]
    acc = xb[0]
    for t in range(1, T):
        acc = acc + xb[t]
    mean_ref[0] = acc * (1.0 / T)
    xt_ref[0] = xb.reshape(D, NB).T


def _mean_transpose(x2):
    return pl.pallas_call(
        _mt_body,
        grid=(B, N // NB),
        in_specs=[pl.BlockSpec((1, T, C, NB), lambda b, n: (b, 0, 0, n))],
        out_specs=[
            pl.BlockSpec((1, NB, D), lambda b, n: (b, n, 0)),
            pl.BlockSpec((1, C, NB), lambda b, n: (b, 0, n)),
        ],
        out_shape=[
            jax.ShapeDtypeStruct((B, N, D), jnp.float32),
            jax.ShapeDtypeStruct((B, C, N), jnp.float32),
        ],
        compiler_params=pltpu.CompilerParams(
            dimension_semantics=("arbitrary", "arbitrary")),
    )(x2)


def _tk_body(s_ref, meta_ref):
    sc = s_ref[...]
    bits = lax.bitcast_convert_type(sc, jnp.int32)
    def step(i, v):
        cand = jnp.bitwise_or(v, jnp.left_shift(jnp.int32(1), 30 - i))
        cnt = jnp.sum((bits >= cand).astype(jnp.int32), axis=1, keepdims=True)
        return jnp.where(cnt >= K, cand, v)
    v = lax.fori_loop(0, 31, step, jnp.zeros((B, 1), jnp.int32))
    cgt = jnp.sum((bits > v).astype(jnp.int32), axis=1, keepdims=True)
    need = K - cgt
    lane = lax.broadcasted_iota(jnp.int32, (B, 32), 1)
    meta_ref[...] = jnp.where(lane < 16,
                              jnp.broadcast_to(v, (B, 32)),
                              jnp.broadcast_to(need, (B, 32)))


def _topk_meta(scores):
    return pl.pallas_call(
        _tk_body,
        in_specs=[pl.BlockSpec((B, N), lambda: (0, 0))],
        out_specs=pl.BlockSpec((B, 32), lambda: (0, 0)),
        out_shape=jax.ShapeDtypeStruct((B, 32), jnp.int32),
    )(scores)


def _sc_body(xt_hbm, sc_hbm, meta_hbm, out_hbm, tidx_hbm,
             sc_v, idx_v, scale_v, meta_v, gidx_v, rows_v, sem):
    cid = lax.axis_index("c")
    sid = lax.axis_index("s")
    wid = sid * 2 + cid
    b = wid // WPB
    q = wid % WPB

    pltpu.sync_copy(sc_hbm.at[pl.ds(b * N, N)], sc_v)
    pltpu.sync_copy(meta_hbm.at[pl.ds(b * 32, 32)], meta_v)
    vvec = meta_v[pl.ds(0, 16)]
    needvec = meta_v[pl.ds(16, 16)]

    def cstep(i, carry):
        off, eqrun = carry
        sv = sc_v[pl.ds(i * 16, 16)]
        sbits = plsc.bitcast(sv, jnp.int32)
        gt = sbits > vvec
        eq = sbits == vvec
        eqc = plsc.cumsum(jnp.where(eq, 1, 0)) + eqrun
        sel = jnp.logical_or(gt, jnp.logical_and(eq, eqc <= needvec))
        idxv = lax.iota(jnp.int32, 16) + i * 16
        plsc.store_compressed(idx_v.at[pl.ds(off, 16)], idxv, mask=sel)
        scl = sv / (sv + EPS)
        plsc.store_compressed(scale_v.at[pl.ds(off, 16)], scl, mask=sel)
        npop = jnp.sum(jnp.where(sel, 1, 0))
        neq = jnp.sum(jnp.where(eq, 1, 0))
        return (off + npop, eqrun + neq)

    lax.fori_loop(0, N // 16, cstep, (jnp.int32(0), jnp.int32(0)))

    @pl.when(q == 0)
    def _():
        pltpu.sync_copy(idx_v.at[pl.ds(0, K)], tidx_hbm.at[b])

    for ch in range(ROWS_PER_W // CHUNK):
        jb = q * ROWS_PER_W + ch * CHUNK
        for v4 in range(CHUNK // 16):
            gidx_v[pl.ds(v4 * 16, 16)] = idx_v[pl.ds(jb + v4 * 16, 16)] + b * N
        pltpu.async_copy(xt_hbm.at[gidx_v], rows_v, sem).wait()

        def rstep(r, _):
            sp = plsc.load_gather(scale_v, [jnp.full((16,), jb + r, jnp.int32)])
            for kk in range(D // 16):
                rows_v[r, pl.ds(kk * 16, 16)] = rows_v[r, pl.ds(kk * 16, 16)] * sp
            return 0
        lax.fori_loop(0, CHUNK, rstep, 0)

        pltpu.sync_copy(rows_v, out_hbm.at[pl.ds(b * K + jb, CHUNK)])


@functools.cache
def _make_sc_gather():
    return functools.partial(
        pl.kernel,
        mesh=plsc.VectorSubcoreMesh(core_axis_name="c", subcore_axis_name="s"),
        out_type=[
            jax.ShapeDtypeStruct((B * K, D), jnp.float32),
            jax.ShapeDtypeStruct((B, K), jnp.int32),
        ],
        scratch_types=[
            pltpu.VMEM((N,), jnp.float32),
            pltpu.VMEM((K + 16,), jnp.int32),
            pltpu.VMEM((K + 16,), jnp.float32),
            pltpu.VMEM((32,), jnp.int32),
            pltpu.VMEM((CHUNK,), jnp.int32),
            pltpu.VMEM((CHUNK, D), jnp.float32),
            pltpu.SemaphoreType.DMA,
        ],
        compiler_params=pltpu.CompilerParams(needs_layout_passes=False),
    )(_sc_body)


def _sc_gather(xt_flat, sc_flat, meta_flat):
    return _make_sc_gather()(xt_flat, sc_flat, meta_flat)


def _router_conv(x, w, b):
    y = lax.conv_general_dilated(x, w, (1, 1), 'SAME',
                                 dimension_numbers=('NCHW', 'OIHW', 'NCHW'))
    return y + b[None, :, None, None]


def kernel(x, W1, b1, W2, b2):
    x2 = x.reshape(B, T, C, N)
    xt, mean = _mean_transpose(x2)
    xm = mean.reshape(B, C, H, W)
    h = _router_conv(xm, W1, b1)
    h = jnp.where(h >= 0, h, 0.01 * h)
    s = _router_conv(h, W2, b2)
    scores = jax.nn.sigmoid(s).reshape(B, N)
    meta = _topk_meta(scores)
    out_flat, top_idx = _sc_gather(xt.reshape(B * N, D),
                                   scores.reshape(B * N),
                                   meta.reshape(B * 32))
    return out_flat.reshape(B, K, T, C), top_idx


# speedup vs baseline: 3.3384x; 1.2970x over previous
import functools

import jax
import jax.numpy as jnp
from jax import lax
from jax.experimental import pallas as pl
from jax.experimental.pallas import tpu as pltpu
from jax.experimental.pallas import tpu_sc as plsc

B, T, C, H, W = 8, 8, 96, 64, 64
N = H * W
K = N // 2
D = T * C
NB = 512
CHUNK = 64
WPB = 4
ROWS_PER_W = K // WPB
EPS = 1e-6


HB = 8


def _mt_body(x_ref, xt_ref, mean_ref):
    v3 = x_ref[0].reshape(T, HB * W, C)
    acc = v3[0]
    for t in range(1, T):
        acc = acc + v3[t]
    mean_ref[0] = (acc * (1.0 / T)).T
    for t in range(T):
        xt_ref[0, :, pl.ds(t * C, C)] = v3[t]


def _mean_transpose(x5):
    return pl.pallas_call(
        _mt_body,
        grid=(B, H // HB),
        in_specs=[pl.BlockSpec((1, T, HB, W, C), lambda b, h: (b, 0, h, 0, 0))],
        out_specs=[
            pl.BlockSpec((1, NB, D), lambda b, h: (b, h, 0)),
            pl.BlockSpec((1, C, NB), lambda b, h: (b, 0, h)),
        ],
        out_shape=[
            jax.ShapeDtypeStruct((B, N, D), jnp.float32),
            jax.ShapeDtypeStruct((B, C, N), jnp.float32),
        ],
        compiler_params=pltpu.CompilerParams(
            dimension_semantics=("arbitrary", "arbitrary")),
    )(x5)


def _tk_body(s_ref, meta_ref):
    sc = s_ref[...]
    bits = lax.bitcast_convert_type(sc, jnp.int32)
    def step(i, v):
        cand = jnp.bitwise_or(v, jnp.left_shift(jnp.int32(1), 30 - i))
        cnt = jnp.sum((bits >= cand).astype(jnp.int32), axis=1, keepdims=True)
        return jnp.where(cnt >= K, cand, v)
    v = lax.fori_loop(0, 31, step, jnp.zeros((B, 1), jnp.int32))
    cgt = jnp.sum((bits > v).astype(jnp.int32), axis=1, keepdims=True)
    need = K - cgt
    lane = lax.broadcasted_iota(jnp.int32, (B, 32), 1)
    meta_ref[...] = jnp.where(lane < 16,
                              jnp.broadcast_to(v, (B, 32)),
                              jnp.broadcast_to(need, (B, 32)))


def _topk_meta(scores):
    return pl.pallas_call(
        _tk_body,
        in_specs=[pl.BlockSpec((B, N), lambda: (0, 0))],
        out_specs=pl.BlockSpec((B, 32), lambda: (0, 0)),
        out_shape=jax.ShapeDtypeStruct((B, 32), jnp.int32),
    )(scores)


def _sc_body(xt_hbm, sc_hbm, meta_hbm, out_hbm, tidx_hbm,
             sc_v, idx_v, scale_v, meta_v, gidx_v, rows_v, sem):
    cid = lax.axis_index("c")
    sid = lax.axis_index("s")
    wid = sid * 2 + cid
    b = wid // WPB
    q = wid % WPB

    pltpu.sync_copy(sc_hbm.at[pl.ds(b * N, N)], sc_v)
    pltpu.sync_copy(meta_hbm.at[pl.ds(b * 32, 32)], meta_v)
    vvec = meta_v[pl.ds(0, 16)]
    needvec = meta_v[pl.ds(16, 16)]

    def cstep(i, carry):
        off, eqrun = carry
        sv = sc_v[pl.ds(i * 16, 16)]
        sbits = plsc.bitcast(sv, jnp.int32)
        gt = sbits > vvec
        eq = sbits == vvec
        eqc = plsc.cumsum(jnp.where(eq, 1, 0)) + eqrun
        sel = jnp.logical_or(gt, jnp.logical_and(eq, eqc <= needvec))
        idxv = lax.iota(jnp.int32, 16) + i * 16
        plsc.store_compressed(idx_v.at[pl.ds(off, 16)], idxv, mask=sel)
        scl = sv / (sv + EPS)
        plsc.store_compressed(scale_v.at[pl.ds(off, 16)], scl, mask=sel)
        npop = jnp.sum(jnp.where(sel, 1, 0))
        neq = jnp.sum(jnp.where(eq, 1, 0))
        return (off + npop, eqrun + neq)

    lax.fori_loop(0, N // 16, cstep, (jnp.int32(0), jnp.int32(0)))

    @pl.when(q == 0)
    def _():
        pltpu.sync_copy(idx_v.at[pl.ds(0, K)], tidx_hbm.at[b])

    for ch in range(ROWS_PER_W // CHUNK):
        jb = q * ROWS_PER_W + ch * CHUNK
        for v4 in range(CHUNK // 16):
            gidx_v[pl.ds(v4 * 16, 16)] = idx_v[pl.ds(jb + v4 * 16, 16)] + b * N
        pltpu.async_copy(xt_hbm.at[gidx_v], rows_v, sem).wait()

        def rstep(r, _):
            sp = plsc.load_gather(scale_v, [jnp.full((16,), jb + r, jnp.int32)])
            for kk in range(D // 16):
                rows_v[r, pl.ds(kk * 16, 16)] = rows_v[r, pl.ds(kk * 16, 16)] * sp
            return 0
        lax.fori_loop(0, CHUNK, rstep, 0)

        pltpu.sync_copy(rows_v, out_hbm.at[pl.ds(b * K + jb, CHUNK)])


@functools.cache
def _make_sc_gather():
    return functools.partial(
        pl.kernel,
        mesh=plsc.VectorSubcoreMesh(core_axis_name="c", subcore_axis_name="s"),
        out_type=[
            jax.ShapeDtypeStruct((B * K, D), jnp.float32),
            jax.ShapeDtypeStruct((B, K), jnp.int32),
        ],
        scratch_types=[
            pltpu.VMEM((N,), jnp.float32),
            pltpu.VMEM((K + 16,), jnp.int32),
            pltpu.VMEM((K + 16,), jnp.float32),
            pltpu.VMEM((32,), jnp.int32),
            pltpu.VMEM((CHUNK,), jnp.int32),
            pltpu.VMEM((CHUNK, D), jnp.float32),
            pltpu.SemaphoreType.DMA,
        ],
        compiler_params=pltpu.CompilerParams(needs_layout_passes=False),
    )(_sc_body)


def _sc_gather(xt_flat, sc_flat, meta_flat):
    return _make_sc_gather()(xt_flat, sc_flat, meta_flat)


def _router_conv(x, w, b):
    y = lax.conv_general_dilated(x, w, (1, 1), 'SAME',
                                 dimension_numbers=('NCHW', 'OIHW', 'NCHW'))
    return y + b[None, :, None, None]


def kernel(x, W1, b1, W2, b2):
    x5 = jnp.transpose(x, (0, 1, 3, 4, 2))
    xt, mean = _mean_transpose(x5)
    xm = mean.reshape(B, C, H, W)
    h = _router_conv(xm, W1, b1)
    h = jnp.where(h >= 0, h, 0.01 * h)
    s = _router_conv(h, W2, b2)
    scores = jax.nn.sigmoid(s).reshape(B, N)
    meta = _topk_meta(scores)
    out_flat, top_idx = _sc_gather(xt.reshape(B * N, D),
                                   scores.reshape(B * N),
                                   meta.reshape(B * 32))
    return out_flat.reshape(B, K, T, C), top_idx


# speedup vs baseline: 4.1405x; 1.2403x over previous
import functools

import jax
import jax.numpy as jnp
from jax import lax
from jax.experimental import pallas as pl
from jax.experimental.pallas import tpu as pltpu
from jax.experimental.pallas import tpu_sc as plsc

B, T, C, H, W = 8, 8, 96, 64, 64
N = H * W
K = N // 2
D = T * C
NB = 512
CHUNK = 64
WPB = 4
ROWS_PER_W = K // WPB
EPS = 1e-6


HB = 8


def _mt_body(x_ref, xt_ref, mean_ref):
    v3 = x_ref[0].reshape(T, HB * W, C)
    acc = v3[0]
    for t in range(1, T):
        acc = acc + v3[t]
    mean_ref[0] = (acc * (1.0 / T)).T
    for t in range(T):
        xt_ref[0, :, pl.ds(t * C, C)] = v3[t]


def _mean_transpose(x5):
    return pl.pallas_call(
        _mt_body,
        grid=(B, H // HB),
        in_specs=[pl.BlockSpec((1, T, HB, W, C), lambda b, h: (b, 0, h, 0, 0))],
        out_specs=[
            pl.BlockSpec((1, NB, D), lambda b, h: (b, h, 0)),
            pl.BlockSpec((1, C, NB), lambda b, h: (b, 0, h)),
        ],
        out_shape=[
            jax.ShapeDtypeStruct((B, N, D), jnp.float32),
            jax.ShapeDtypeStruct((B, C, N), jnp.float32),
        ],
        compiler_params=pltpu.CompilerParams(
            dimension_semantics=("arbitrary", "arbitrary")),
    )(x5)


def _tk_body(s_ref, meta_ref):
    sc = s_ref[...]
    bits = lax.bitcast_convert_type(sc, jnp.int32)
    def step(i, v):
        cand = jnp.bitwise_or(v, jnp.left_shift(jnp.int32(1), 30 - i))
        cnt = jnp.sum((bits >= cand).astype(jnp.int32), axis=1, keepdims=True)
        return jnp.where(cnt >= K, cand, v)
    v = lax.fori_loop(0, 31, step, jnp.zeros((B, 1), jnp.int32))
    cgt = jnp.sum((bits > v).astype(jnp.int32), axis=1, keepdims=True)
    need = K - cgt
    lane = lax.broadcasted_iota(jnp.int32, (B, 32), 1)
    meta_ref[...] = jnp.where(lane < 16,
                              jnp.broadcast_to(v, (B, 32)),
                              jnp.broadcast_to(need, (B, 32)))


def _topk_meta(scores):
    return pl.pallas_call(
        _tk_body,
        in_specs=[pl.BlockSpec((B, N), lambda: (0, 0))],
        out_specs=pl.BlockSpec((B, 32), lambda: (0, 0)),
        out_shape=jax.ShapeDtypeStruct((B, 32), jnp.int32),
    )(scores)


def _sc_body(xt_hbm, sc_hbm, meta_hbm, out_hbm, tidx_hbm,
             sc_v, idx_v, scale_v, meta_v, gidx_v, rows_v, sem):
    cid = lax.axis_index("c")
    sid = lax.axis_index("s")
    wid = sid * 2 + cid
    b = wid // WPB
    q = wid % WPB

    pltpu.sync_copy(sc_hbm.at[pl.ds(b * N, N)], sc_v)
    pltpu.sync_copy(meta_hbm.at[pl.ds(b * 32, 32)], meta_v)
    vvec = meta_v[pl.ds(0, 16)]
    needvec = meta_v[pl.ds(16, 16)]

    def cstep(i, carry):
        off, eqrun = carry
        sv = sc_v[pl.ds(i * 16, 16)]
        sbits = plsc.bitcast(sv, jnp.int32)
        gt = sbits > vvec
        eq = sbits == vvec
        eqc = plsc.cumsum(jnp.where(eq, 1, 0)) + eqrun
        sel = jnp.logical_or(gt, jnp.logical_and(eq, eqc <= needvec))
        idxv = lax.iota(jnp.int32, 16) + i * 16
        plsc.store_compressed(idx_v.at[pl.ds(off, 16)], idxv, mask=sel)
        scl = sv / (sv + EPS)
        plsc.store_compressed(scale_v.at[pl.ds(off, 16)], scl, mask=sel)
        npop = jnp.sum(jnp.where(sel, 1, 0))
        neq = jnp.sum(jnp.where(eq, 1, 0))
        return (off + npop, eqrun + neq)

    lax.fori_loop(0, N // 16, cstep, (jnp.int32(0), jnp.int32(0)))

    @pl.when(q == 0)
    def _():
        pltpu.sync_copy(idx_v.at[pl.ds(0, K)], tidx_hbm.at[b])

    for ch in range(ROWS_PER_W // CHUNK):
        jb = q * ROWS_PER_W + ch * CHUNK
        for v4 in range(CHUNK // 16):
            gidx_v[pl.ds(v4 * 16, 16)] = idx_v[pl.ds(jb + v4 * 16, 16)] + b * N
        pltpu.async_copy(xt_hbm.at[gidx_v], rows_v, sem).wait()

        def rstep(r, _):
            sp = plsc.load_gather(scale_v, [jnp.full((16,), jb + r, jnp.int32)])
            for kk in range(D // 16):
                rows_v[r, pl.ds(kk * 16, 16)] = rows_v[r, pl.ds(kk * 16, 16)] * sp
            return 0
        lax.fori_loop(0, CHUNK, rstep, 0)

        pltpu.sync_copy(rows_v, out_hbm.at[pl.ds(b * K + jb, CHUNK)])


@functools.cache
def _make_sc_gather():
    return functools.partial(
        pl.kernel,
        mesh=plsc.VectorSubcoreMesh(core_axis_name="c", subcore_axis_name="s"),
        out_type=[
            jax.ShapeDtypeStruct((B * K, D), jnp.float32),
            jax.ShapeDtypeStruct((B, K), jnp.int32),
        ],
        scratch_types=[
            pltpu.VMEM((N,), jnp.float32),
            pltpu.VMEM((K + 16,), jnp.int32),
            pltpu.VMEM((K + 16,), jnp.float32),
            pltpu.VMEM((32,), jnp.int32),
            pltpu.VMEM((CHUNK,), jnp.int32),
            pltpu.VMEM((CHUNK, D), jnp.float32),
            pltpu.SemaphoreType.DMA,
        ],
        compiler_params=pltpu.CompilerParams(needs_layout_passes=False),
    )(_sc_body)


def _sc_gather(xt_flat, sc_flat, meta_flat):
    return _make_sc_gather()(xt_flat, sc_flat, meta_flat)


KB = 512


def _ot_body(in_ref, out_ref):
    vt = in_ref[...].T
    out_ref[0] = vt.reshape(T, C, KB)


def _out_transpose(out_flat):
    return pl.pallas_call(
        _ot_body,
        grid=(B, K // KB),
        in_specs=[pl.BlockSpec((KB, D), lambda b, k: (b * (K // KB) + k, 0))],
        out_specs=pl.BlockSpec((1, T, C, KB), lambda b, k: (b, 0, 0, k)),
        out_shape=jax.ShapeDtypeStruct((B, T, C, K), jnp.float32),
        compiler_params=pltpu.CompilerParams(
            dimension_semantics=("arbitrary", "arbitrary")),
    )(out_flat)


def _router_conv(x, w, b):
    y = lax.conv_general_dilated(x, w, (1, 1), 'SAME',
                                 dimension_numbers=('NCHW', 'OIHW', 'NCHW'))
    return y + b[None, :, None, None]


def kernel(x, W1, b1, W2, b2):
    x5 = jnp.transpose(x, (0, 1, 3, 4, 2))
    xt, mean = _mean_transpose(x5)
    xm = mean.reshape(B, C, H, W)
    h = _router_conv(xm, W1, b1)
    h = jnp.where(h >= 0, h, 0.01 * h)
    s = _router_conv(h, W2, b2)
    scores = jax.nn.sigmoid(s).reshape(B, N)
    meta = _topk_meta(scores)
    out_flat, top_idx = _sc_gather(xt.reshape(B * N, D),
                                   scores.reshape(B * N),
                                   meta.reshape(B * 32))
    out_tck = _out_transpose(out_flat)
    return jnp.transpose(out_tck, (0, 3, 1, 2)), top_idx


# speedup vs baseline: 4.4057x; 1.0641x over previous
import functools

import jax
import jax.numpy as jnp
from jax import lax
from jax.experimental import pallas as pl
from jax.experimental.pallas import tpu as pltpu
from jax.experimental.pallas import tpu_sc as plsc

B, T, C, H, W = 8, 8, 96, 64, 64
N = H * W
K = N // 2
D = T * C
NB = 512
CHUNK = 64
WPB = 4
ROWS_PER_W = K // WPB
EPS = 1e-6


HB = 8


def _mt_body(x_ref, xt_ref, mean_ref):
    v3 = x_ref[0].reshape(T, HB * W, C)
    acc = v3[0]
    for t in range(1, T):
        acc = acc + v3[t]
    mean_ref[0] = (acc * (1.0 / T)).T
    for t in range(T):
        xt_ref[0, :, pl.ds(t * C, C)] = v3[t]


def _mean_transpose(x5):
    return pl.pallas_call(
        _mt_body,
        grid=(B, H // HB),
        in_specs=[pl.BlockSpec((1, T, HB, W, C), lambda b, h: (b, 0, h, 0, 0))],
        out_specs=[
            pl.BlockSpec((1, NB, D), lambda b, h: (b, h, 0)),
            pl.BlockSpec((1, C, NB), lambda b, h: (b, 0, h)),
        ],
        out_shape=[
            jax.ShapeDtypeStruct((B, N, D), jnp.float32),
            jax.ShapeDtypeStruct((B, C, N), jnp.float32),
        ],
        compiler_params=pltpu.CompilerParams(
            dimension_semantics=("arbitrary", "arbitrary")),
    )(x5)


def _tk_body(s_ref, meta_ref):
    sc = s_ref[...]
    bits = lax.bitcast_convert_type(sc, jnp.int32)
    def step(i, v):
        cand = jnp.bitwise_or(v, jnp.left_shift(jnp.int32(1), 30 - i))
        cnt = jnp.sum((bits >= cand).astype(jnp.int32), axis=1, keepdims=True)
        return jnp.where(cnt >= K, cand, v)
    v = lax.fori_loop(0, 31, step, jnp.zeros((B, 1), jnp.int32))
    cgt = jnp.sum((bits > v).astype(jnp.int32), axis=1, keepdims=True)
    need = K - cgt
    lane = lax.broadcasted_iota(jnp.int32, (B, 32), 1)
    meta_ref[...] = jnp.where(lane < 16,
                              jnp.broadcast_to(v, (B, 32)),
                              jnp.broadcast_to(need, (B, 32)))


def _topk_meta(scores):
    return pl.pallas_call(
        _tk_body,
        in_specs=[pl.BlockSpec((B, N), lambda: (0, 0))],
        out_specs=pl.BlockSpec((B, 32), lambda: (0, 0)),
        out_shape=jax.ShapeDtypeStruct((B, 32), jnp.int32),
    )(scores)


def _sc_body(xt_hbm, sc_hbm, meta_hbm, out_hbm, tidx_hbm,
             sc_v, idx_v, scale_v, meta_v, gidx_v, rows_v, sem):
    cid = lax.axis_index("c")
    sid = lax.axis_index("s")
    wid = sid * 2 + cid
    b = wid // WPB
    q = wid % WPB

    pltpu.sync_copy(sc_hbm.at[pl.ds(b * N, N)], sc_v)
    pltpu.sync_copy(meta_hbm.at[pl.ds(b * 32, 32)], meta_v)
    vvec = meta_v[pl.ds(0, 16)]
    needvec = meta_v[pl.ds(16, 16)]

    stop = (q + 1) * ROWS_PER_W

    def ccond(carry):
        i, off, eqrun = carry
        return jnp.logical_and(i < N // 16, off < stop)

    def cstep(carry):
        i, off, eqrun = carry
        sv = sc_v[pl.ds(i * 16, 16)]
        sbits = plsc.bitcast(sv, jnp.int32)
        gt = sbits > vvec
        eq = sbits == vvec
        eqc = plsc.cumsum(jnp.where(eq, 1, 0)) + eqrun
        sel = jnp.logical_or(gt, jnp.logical_and(eq, eqc <= needvec))
        idxv = lax.iota(jnp.int32, 16) + i * 16
        plsc.store_compressed(idx_v.at[pl.ds(off, 16)], idxv, mask=sel)
        scl = sv / (sv + EPS)
        plsc.store_compressed(scale_v.at[pl.ds(off, 16)], scl, mask=sel)
        npop = jnp.sum(jnp.where(sel, 1, 0))
        neq = jnp.sum(jnp.where(eq, 1, 0))
        return (i + 1, off + npop, eqrun + neq)

    lax.while_loop(ccond, cstep, (jnp.int32(0), jnp.int32(0), jnp.int32(0)))

    @pl.when(q == WPB - 1)
    def _():
        pltpu.sync_copy(idx_v.at[pl.ds(0, K)], tidx_hbm.at[b])

    NCH = ROWS_PER_W // CHUNK

    def fire(ch, buf):
        jb = q * ROWS_PER_W + ch * CHUNK
        for v4 in range(CHUNK // 16):
            gidx_v[pl.ds(buf * CHUNK + v4 * 16, 16)] = (
                idx_v[pl.ds(jb + v4 * 16, 16)] + b * N)
        return pltpu.async_copy(
            xt_hbm.at[gidx_v.at[pl.ds(buf * CHUNK, CHUNK)]],
            rows_v.at[buf], sem)

    cp = fire(0, 0)
    for ch in range(NCH):
        buf = ch % 2
        cp.wait()
        if ch + 1 < NCH:
            cp = fire(ch + 1, 1 - buf)
        jb = q * ROWS_PER_W + ch * CHUNK

        def rstep(r, _):
            sp = plsc.load_gather(scale_v, [jnp.full((16,), jb + r, jnp.int32)])
            for kk in range(D // 16):
                rows_v[buf, r, pl.ds(kk * 16, 16)] = (
                    rows_v[buf, r, pl.ds(kk * 16, 16)] * sp)
            return 0
        lax.fori_loop(0, CHUNK, rstep, 0)

        pltpu.sync_copy(rows_v.at[buf], out_hbm.at[pl.ds(b * K + jb, CHUNK)])


@functools.cache
def _make_sc_gather():
    return functools.partial(
        pl.kernel,
        mesh=plsc.VectorSubcoreMesh(core_axis_name="c", subcore_axis_name="s"),
        out_type=[
            jax.ShapeDtypeStruct((B * K, D), jnp.float32),
            jax.ShapeDtypeStruct((B, K), jnp.int32),
        ],
        scratch_types=[
            pltpu.VMEM((N,), jnp.float32),
            pltpu.VMEM((K + 16,), jnp.int32),
            pltpu.VMEM((K + 16,), jnp.float32),
            pltpu.VMEM((32,), jnp.int32),
            pltpu.VMEM((2 * CHUNK,), jnp.int32),
            pltpu.VMEM((2, CHUNK, D), jnp.float32),
            pltpu.SemaphoreType.DMA,
        ],
        compiler_params=pltpu.CompilerParams(needs_layout_passes=False),
    )(_sc_body)


def _sc_gather(xt_flat, sc_flat, meta_flat):
    return _make_sc_gather()(xt_flat, sc_flat, meta_flat)


KB = 512


def _ot_body(in_ref, out_ref):
    vt = in_ref[...].T
    out_ref[0] = vt.reshape(T, C, KB)


def _out_transpose(out_flat):
    return pl.pallas_call(
        _ot_body,
        grid=(B, K // KB),
        in_specs=[pl.BlockSpec((KB, D), lambda b, k: (b * (K // KB) + k, 0))],
        out_specs=pl.BlockSpec((1, T, C, KB), lambda b, k: (b, 0, 0, k)),
        out_shape=jax.ShapeDtypeStruct((B, T, C, K), jnp.float32),
        compiler_params=pltpu.CompilerParams(
            dimension_semantics=("arbitrary", "arbitrary")),
    )(out_flat)


def _router_conv(x, w, b):
    y = lax.conv_general_dilated(x, w, (1, 1), 'SAME',
                                 dimension_numbers=('NCHW', 'OIHW', 'NCHW'))
    return y + b[None, :, None, None]


def kernel(x, W1, b1, W2, b2):
    x5 = jnp.transpose(x, (0, 1, 3, 4, 2))
    xt, mean = _mean_transpose(x5)
    xm = mean.reshape(B, C, H, W)
    h = _router_conv(xm, W1, b1)
    h = jnp.where(h >= 0, h, 0.01 * h)
    s = _router_conv(h, W2, b2)
    scores = jax.nn.sigmoid(s).reshape(B, N)
    meta = _topk_meta(scores)
    out_flat, top_idx = _sc_gather(xt.reshape(B * N, D),
                                   scores.reshape(B * N),
                                   meta.reshape(B * 32))
    out_tck = _out_transpose(out_flat)
    return jnp.transpose(out_tck, (0, 3, 1, 2)), top_idx


# speedup vs baseline: 4.8617x; 1.1035x over previous
import functools

import jax
import jax.numpy as jnp
from jax import lax
from jax.experimental import pallas as pl
from jax.experimental.pallas import tpu as pltpu
from jax.experimental.pallas import tpu_sc as plsc

B, T, C, H, W = 8, 8, 96, 64, 64
N = H * W
K = N // 2
D = T * C
NB = 1024
CHUNK = 64
WPB = 4
ROWS_PER_W = K // WPB
EPS = 1e-6


HB = 16


def _mt_body(x_ref, xt_ref, mean_ref):
    v3 = x_ref[0].reshape(T, HB * W, C)
    acc = v3[0]
    for t in range(1, T):
        acc = acc + v3[t]
    mean_ref[0] = (acc * (1.0 / T)).T
    for t in range(T):
        xt_ref[0, :, pl.ds(t * C, C)] = v3[t]


def _mean_transpose(x5):
    return pl.pallas_call(
        _mt_body,
        grid=(B, H // HB),
        in_specs=[pl.BlockSpec((1, T, HB, W, C), lambda b, h: (b, 0, h, 0, 0))],
        out_specs=[
            pl.BlockSpec((1, NB, D), lambda b, h: (b, h, 0)),
            pl.BlockSpec((1, C, NB), lambda b, h: (b, 0, h)),
        ],
        out_shape=[
            jax.ShapeDtypeStruct((B, N, D), jnp.float32),
            jax.ShapeDtypeStruct((B, C, N), jnp.float32),
        ],
        compiler_params=pltpu.CompilerParams(
            dimension_semantics=("arbitrary", "arbitrary")),
    )(x5)


def _tk_body(s_ref, meta_ref):
    sc = s_ref[...]
    bits = lax.bitcast_convert_type(sc, jnp.int32)
    def step(i, v):
        cand = jnp.bitwise_or(v, jnp.left_shift(jnp.int32(1), 30 - i))
        cnt = jnp.sum((bits >= cand).astype(jnp.int32), axis=1, keepdims=True)
        return jnp.where(cnt >= K, cand, v)
    v = lax.fori_loop(0, 31, step, jnp.zeros((B, 1), jnp.int32))
    cgt = jnp.sum((bits > v).astype(jnp.int32), axis=1, keepdims=True)
    need = K - cgt
    lane = lax.broadcasted_iota(jnp.int32, (B, 32), 1)
    meta_ref[...] = jnp.where(lane < 16,
                              jnp.broadcast_to(v, (B, 32)),
                              jnp.broadcast_to(need, (B, 32)))


def _topk_meta(scores):
    return pl.pallas_call(
        _tk_body,
        in_specs=[pl.BlockSpec((B, N), lambda: (0, 0))],
        out_specs=pl.BlockSpec((B, 32), lambda: (0, 0)),
        out_shape=jax.ShapeDtypeStruct((B, 32), jnp.int32),
    )(scores)


def _sc_body(xt_hbm, sc_hbm, meta_hbm, out_hbm, tidx_hbm,
             sc_v, idx_v, scale_v, meta_v, gidx_v, rows_v, sem, wsem):
    cid = lax.axis_index("c")
    sid = lax.axis_index("s")
    wid = sid * 2 + cid
    b = wid // WPB
    q = wid % WPB

    pltpu.sync_copy(sc_hbm.at[pl.ds(b * N, N)], sc_v)
    pltpu.sync_copy(meta_hbm.at[pl.ds(b * 32, 32)], meta_v)
    vvec = meta_v[pl.ds(0, 16)]
    needvec = meta_v[pl.ds(16, 16)]

    stop = (q + 1) * ROWS_PER_W

    def ccond(carry):
        i, off, eqrun = carry
        return jnp.logical_and(i < N // 16, off < stop)

    def cstep(carry):
        i, off, eqrun = carry
        sv = sc_v[pl.ds(i * 16, 16)]
        sbits = plsc.bitcast(sv, jnp.int32)
        gt = sbits > vvec
        eq = sbits == vvec
        eqc = plsc.cumsum(jnp.where(eq, 1, 0)) + eqrun
        sel = jnp.logical_or(gt, jnp.logical_and(eq, eqc <= needvec))
        idxv = lax.iota(jnp.int32, 16) + i * 16
        plsc.store_compressed(idx_v.at[pl.ds(off, 16)], idxv, mask=sel)
        scl = sv / (sv + EPS)
        plsc.store_compressed(scale_v.at[pl.ds(off, 16)], scl, mask=sel)
        npop = jnp.sum(jnp.where(sel, 1, 0))
        neq = jnp.sum(jnp.where(eq, 1, 0))
        return (i + 1, off + npop, eqrun + neq)

    lax.while_loop(ccond, cstep, (jnp.int32(0), jnp.int32(0), jnp.int32(0)))

    @pl.when(q == WPB - 1)
    def _():
        pltpu.sync_copy(idx_v.at[pl.ds(0, K)], tidx_hbm.at[b])

    NCH = ROWS_PER_W // CHUNK

    def fire(ch, buf):
        jb = q * ROWS_PER_W + ch * CHUNK
        for v4 in range(CHUNK // 16):
            gidx_v[pl.ds(buf * CHUNK + v4 * 16, 16)] = (
                idx_v[pl.ds(jb + v4 * 16, 16)] + b * N)
        return pltpu.async_copy(
            xt_hbm.at[gidx_v.at[pl.ds(buf * CHUNK, CHUNK)]],
            rows_v.at[buf], sem)

    cp = fire(0, 0)
    wcp = [None, None]
    for ch in range(NCH):
        buf = ch % 2
        cp.wait()
        if ch + 1 < NCH:
            if wcp[1 - buf] is not None:
                wcp[1 - buf].wait()
                wcp[1 - buf] = None
            cp = fire(ch + 1, 1 - buf)
        jb = q * ROWS_PER_W + ch * CHUNK

        def rstep(r, _):
            sp = plsc.load_gather(scale_v, [jnp.full((16,), jb + r, jnp.int32)])
            for kk in range(D // 16):
                rows_v[buf, r, pl.ds(kk * 16, 16)] = (
                    rows_v[buf, r, pl.ds(kk * 16, 16)] * sp)
            return 0
        lax.fori_loop(0, CHUNK, rstep, 0)

        wcp[buf] = pltpu.async_copy(
            rows_v.at[buf], out_hbm.at[pl.ds(b * K + jb, CHUNK)], wsem[buf])
    for w in wcp:
        if w is not None:
            w.wait()


@functools.cache
def _make_sc_gather():
    return functools.partial(
        pl.kernel,
        mesh=plsc.VectorSubcoreMesh(core_axis_name="c", subcore_axis_name="s"),
        out_type=[
            jax.ShapeDtypeStruct((B * K, D), jnp.float32),
            jax.ShapeDtypeStruct((B, K), jnp.int32),
        ],
        scratch_types=[
            pltpu.VMEM((N,), jnp.float32),
            pltpu.VMEM((K + 16,), jnp.int32),
            pltpu.VMEM((K + 16,), jnp.float32),
            pltpu.VMEM((32,), jnp.int32),
            pltpu.VMEM((2 * CHUNK,), jnp.int32),
            pltpu.VMEM((2, CHUNK, D), jnp.float32),
            pltpu.SemaphoreType.DMA,
            (pltpu.SemaphoreType.DMA, pltpu.SemaphoreType.DMA),
        ],
        compiler_params=pltpu.CompilerParams(needs_layout_passes=False),
    )(_sc_body)


def _sc_gather(xt_flat, sc_flat, meta_flat):
    return _make_sc_gather()(xt_flat, sc_flat, meta_flat)


KB = 1024


def _ot_body(in_ref, out_ref):
    vt = in_ref[...].T
    out_ref[0] = vt.reshape(T, C, KB)


def _out_transpose(out_flat):
    return pl.pallas_call(
        _ot_body,
        grid=(B, K // KB),
        in_specs=[pl.BlockSpec((KB, D), lambda b, k: (b * (K // KB) + k, 0))],
        out_specs=pl.BlockSpec((1, T, C, KB), lambda b, k: (b, 0, 0, k)),
        out_shape=jax.ShapeDtypeStruct((B, T, C, K), jnp.float32),
        compiler_params=pltpu.CompilerParams(
            dimension_semantics=("arbitrary", "arbitrary")),
    )(out_flat)


def _router_conv(x, w, b):
    y = lax.conv_general_dilated(x, w, (1, 1), 'SAME',
                                 dimension_numbers=('NCHW', 'OIHW', 'NCHW'))
    return y + b[None, :, None, None]


def kernel(x, W1, b1, W2, b2):
    x5 = jnp.transpose(x, (0, 1, 3, 4, 2))
    xt, mean = _mean_transpose(x5)
    xm = mean.reshape(B, C, H, W)
    h = _router_conv(xm, W1, b1)
    h = jnp.where(h >= 0, h, 0.01 * h)
    s = _router_conv(h, W2, b2)
    scores = jax.nn.sigmoid(s).reshape(B, N)
    meta = _topk_meta(scores)
    out_flat, top_idx = _sc_gather(xt.reshape(B * N, D),
                                   scores.reshape(B * N),
                                   meta.reshape(B * 32))
    out_tck = _out_transpose(out_flat)
    return jnp.transpose(out_tck, (0, 3, 1, 2)), top_idx


# speedup vs baseline: 4.9865x; 1.0257x over previous
import functools

import jax
import jax.numpy as jnp
from jax import lax
from jax.experimental import pallas as pl
from jax.experimental.pallas import tpu as pltpu
from jax.experimental.pallas import tpu_sc as plsc

B, T, C, H, W = 8, 8, 96, 64, 64
N = H * W
K = N // 2
D = T * C
NB = 2048
CHUNK = 64
WPB = 4
ROWS_PER_W = K // WPB
EPS = 1e-6


HB = 32


def _mt_body(x_ref, xt_ref, mean_ref):
    v3 = x_ref[0].reshape(T, HB * W, C)
    acc = v3[0]
    for t in range(1, T):
        acc = acc + v3[t]
    mean_ref[0] = (acc * (1.0 / T)).T
    for t in range(T):
        xt_ref[0, :, pl.ds(t * C, C)] = v3[t]


def _mean_transpose(x5):
    return pl.pallas_call(
        _mt_body,
        grid=(B, H // HB),
        in_specs=[pl.BlockSpec((1, T, HB, W, C), lambda b, h: (b, 0, h, 0, 0))],
        out_specs=[
            pl.BlockSpec((1, NB, D), lambda b, h: (b, h, 0)),
            pl.BlockSpec((1, C, NB), lambda b, h: (b, 0, h)),
        ],
        out_shape=[
            jax.ShapeDtypeStruct((B, N, D), jnp.float32),
            jax.ShapeDtypeStruct((B, C, N), jnp.float32),
        ],
        compiler_params=pltpu.CompilerParams(
            dimension_semantics=("arbitrary", "arbitrary")),
    )(x5)


def _tk_body(s_ref, meta_ref):
    sc = s_ref[...]
    bits = lax.bitcast_convert_type(sc, jnp.int32)
    def step(i, v):
        cand = jnp.bitwise_or(v, jnp.left_shift(jnp.int32(1), 30 - i))
        cnt = jnp.sum((bits >= cand).astype(jnp.int32), axis=1, keepdims=True)
        return jnp.where(cnt >= K, cand, v)
    v = lax.fori_loop(0, 31, step, jnp.zeros((B, 1), jnp.int32))
    cgt = jnp.sum((bits > v).astype(jnp.int32), axis=1, keepdims=True)
    need = K - cgt
    lane = lax.broadcasted_iota(jnp.int32, (B, 32), 1)
    meta_ref[...] = jnp.where(lane < 16,
                              jnp.broadcast_to(v, (B, 32)),
                              jnp.broadcast_to(need, (B, 32)))


def _topk_meta(scores):
    return pl.pallas_call(
        _tk_body,
        in_specs=[pl.BlockSpec((B, N), lambda: (0, 0))],
        out_specs=pl.BlockSpec((B, 32), lambda: (0, 0)),
        out_shape=jax.ShapeDtypeStruct((B, 32), jnp.int32),
    )(scores)


def _sc_body(xt_hbm, sc_hbm, meta_hbm, out_hbm, tidx_hbm,
             sc_v, idx_v, scale_v, meta_v, gidx_v, rows_v, sem, wsem):
    cid = lax.axis_index("c")
    sid = lax.axis_index("s")
    wid = sid * 2 + cid
    b = wid // WPB
    q = wid % WPB

    pltpu.sync_copy(sc_hbm.at[pl.ds(b * N, N)], sc_v)
    pltpu.sync_copy(meta_hbm.at[pl.ds(b * 32, 32)], meta_v)
    vvec = meta_v[pl.ds(0, 16)]
    needvec = meta_v[pl.ds(16, 16)]

    stop = (q + 1) * ROWS_PER_W

    def ccond(carry):
        i, off, eqrun = carry
        return jnp.logical_and(i < N // 16, off < stop)

    def cstep(carry):
        i, off, eqrun = carry
        sv = sc_v[pl.ds(i * 16, 16)]
        sbits = plsc.bitcast(sv, jnp.int32)
        gt = sbits > vvec
        eq = sbits == vvec
        eqc = plsc.cumsum(jnp.where(eq, 1, 0)) + eqrun
        sel = jnp.logical_or(gt, jnp.logical_and(eq, eqc <= needvec))
        idxv = lax.iota(jnp.int32, 16) + i * 16
        plsc.store_compressed(idx_v.at[pl.ds(off, 16)], idxv, mask=sel)
        scl = sv / (sv + EPS)
        plsc.store_compressed(scale_v.at[pl.ds(off, 16)], scl, mask=sel)
        npop = jnp.sum(jnp.where(sel, 1, 0))
        neq = jnp.sum(jnp.where(eq, 1, 0))
        return (i + 1, off + npop, eqrun + neq)

    lax.while_loop(ccond, cstep, (jnp.int32(0), jnp.int32(0), jnp.int32(0)))

    @pl.when(q == WPB - 1)
    def _():
        pltpu.sync_copy(idx_v.at[pl.ds(0, K)], tidx_hbm.at[b])

    NCH = ROWS_PER_W // CHUNK

    def fire(ch, buf):
        jb = q * ROWS_PER_W + ch * CHUNK
        for v4 in range(CHUNK // 16):
            gidx_v[pl.ds(buf * CHUNK + v4 * 16, 16)] = (
                idx_v[pl.ds(jb + v4 * 16, 16)] + b * N)
        return pltpu.async_copy(
            xt_hbm.at[gidx_v.at[pl.ds(buf * CHUNK, CHUNK)]],
            rows_v.at[buf], sem)

    cp = fire(0, 0)
    wcp = [None, None]
    for ch in range(NCH):
        buf = ch % 2
        cp.wait()
        if ch + 1 < NCH:
            if wcp[1 - buf] is not None:
                wcp[1 - buf].wait()
                wcp[1 - buf] = None
            cp = fire(ch + 1, 1 - buf)
        jb = q * ROWS_PER_W + ch * CHUNK

        def rstep(r, _):
            sp = plsc.load_gather(scale_v, [jnp.full((16,), jb + r, jnp.int32)])
            for kk in range(D // 16):
                rows_v[buf, r, pl.ds(kk * 16, 16)] = (
                    rows_v[buf, r, pl.ds(kk * 16, 16)] * sp)
            return 0
        lax.fori_loop(0, CHUNK, rstep, 0)

        wcp[buf] = pltpu.async_copy(
            rows_v.at[buf], out_hbm.at[pl.ds(b * K + jb, CHUNK)], wsem[buf])
    for w in wcp:
        if w is not None:
            w.wait()


@functools.cache
def _make_sc_gather():
    return functools.partial(
        pl.kernel,
        mesh=plsc.VectorSubcoreMesh(core_axis_name="c", subcore_axis_name="s"),
        out_type=[
            jax.ShapeDtypeStruct((B * K, D), jnp.float32),
            jax.ShapeDtypeStruct((B, K), jnp.int32),
        ],
        scratch_types=[
            pltpu.VMEM((N,), jnp.float32),
            pltpu.VMEM((K + 16,), jnp.int32),
            pltpu.VMEM((K + 16,), jnp.float32),
            pltpu.VMEM((32,), jnp.int32),
            pltpu.VMEM((2 * CHUNK,), jnp.int32),
            pltpu.VMEM((2, CHUNK, D), jnp.float32),
            pltpu.SemaphoreType.DMA,
            (pltpu.SemaphoreType.DMA, pltpu.SemaphoreType.DMA),
        ],
        compiler_params=pltpu.CompilerParams(needs_layout_passes=False),
    )(_sc_body)


def _sc_gather(xt_flat, sc_flat, meta_flat):
    return _make_sc_gather()(xt_flat, sc_flat, meta_flat)


KB = 2048


def _ot_body(in_ref, out_ref):
    vt = in_ref[...].T
    out_ref[0] = vt.reshape(T, C, KB)


def _out_transpose(out_flat):
    return pl.pallas_call(
        _ot_body,
        grid=(B, K // KB),
        in_specs=[pl.BlockSpec((KB, D), lambda b, k: (b * (K // KB) + k, 0))],
        out_specs=pl.BlockSpec((1, T, C, KB), lambda b, k: (b, 0, 0, k)),
        out_shape=jax.ShapeDtypeStruct((B, T, C, K), jnp.float32),
        compiler_params=pltpu.CompilerParams(
            dimension_semantics=("arbitrary", "arbitrary")),
    )(out_flat)


def _router_conv(x, w, b):
    y = lax.conv_general_dilated(x, w, (1, 1), 'SAME',
                                 dimension_numbers=('NCHW', 'OIHW', 'NCHW'))
    return y + b[None, :, None, None]


def kernel(x, W1, b1, W2, b2):
    x5 = jnp.transpose(x, (0, 1, 3, 4, 2))
    xt, mean = _mean_transpose(x5)
    xm = mean.reshape(B, C, H, W)
    h = _router_conv(xm, W1, b1)
    h = jnp.where(h >= 0, h, 0.01 * h)
    s = _router_conv(h, W2, b2)
    scores = jax.nn.sigmoid(s).reshape(B, N)
    meta = _topk_meta(scores)
    out_flat, top_idx = _sc_gather(xt.reshape(B * N, D),
                                   scores.reshape(B * N),
                                   meta.reshape(B * 32))
    out_tck = _out_transpose(out_flat)
    return jnp.transpose(out_tck, (0, 3, 1, 2)), top_idx


# speedup vs baseline: 5.0310x; 1.0089x over previous
import functools

import jax
import jax.numpy as jnp
from jax import lax
from jax.experimental import pallas as pl
from jax.experimental.pallas import tpu as pltpu
from jax.experimental.pallas import tpu_sc as plsc

B, T, C, H, W = 8, 8, 96, 64, 64
N = H * W
K = N // 2
D = T * C
NB = 2048
CHUNK = 64
WPB = 4
ROWS_PER_W = K // WPB
EPS = 1e-6


HB = 32


def _mt_body(x_ref, xt_ref, mean_ref):
    v3 = x_ref[0].reshape(T, HB * W, C)
    acc = v3[0]
    for t in range(1, T):
        acc = acc + v3[t]
    mean_ref[0] = (acc * (1.0 / T)).T
    for t in range(T):
        xt_ref[0, :, pl.ds(t * C, C)] = v3[t]


def _mean_transpose(x5):
    return pl.pallas_call(
        _mt_body,
        grid=(B, H // HB),
        in_specs=[pl.BlockSpec((1, T, HB, W, C), lambda b, h: (b, 0, h, 0, 0))],
        out_specs=[
            pl.BlockSpec((1, NB, D), lambda b, h: (b, h, 0)),
            pl.BlockSpec((1, C, NB), lambda b, h: (b, 0, h)),
        ],
        out_shape=[
            jax.ShapeDtypeStruct((B, N, D), jnp.float32),
            jax.ShapeDtypeStruct((B, C, N), jnp.float32),
        ],
        compiler_params=pltpu.CompilerParams(
            dimension_semantics=("arbitrary", "arbitrary")),
    )(x5)


def _tk_body(s_ref, meta_ref):
    sc = s_ref[...]
    bits = lax.bitcast_convert_type(sc, jnp.int32)
    def step(i, v):
        cand = jnp.bitwise_or(v, jnp.left_shift(jnp.int32(1), 30 - i))
        cnt = jnp.sum((bits >= cand).astype(jnp.int32), axis=1, keepdims=True)
        return jnp.where(cnt >= K, cand, v)
    v = lax.fori_loop(0, 31, step, jnp.zeros((B, 1), jnp.int32))
    cgt = jnp.sum((bits > v).astype(jnp.int32), axis=1, keepdims=True)
    need = K - cgt
    lane = lax.broadcasted_iota(jnp.int32, (B, 32), 1)
    meta_ref[...] = jnp.where(lane < 16,
                              jnp.broadcast_to(v, (B, 32)),
                              jnp.broadcast_to(need, (B, 32)))


def _topk_meta(scores):
    return pl.pallas_call(
        _tk_body,
        in_specs=[pl.BlockSpec((B, N), lambda: (0, 0))],
        out_specs=pl.BlockSpec((B, 32), lambda: (0, 0)),
        out_shape=jax.ShapeDtypeStruct((B, 32), jnp.int32),
    )(scores)


def _sc_body(xt_hbm, sc_hbm, meta_hbm, out_hbm, tidx_hbm, scl_hbm,
             sc_v, idx_v, scale_v, meta_v, gidx_v, rows_v, sem, wsem):
    cid = lax.axis_index("c")
    sid = lax.axis_index("s")
    wid = sid * 2 + cid
    b = wid // WPB
    q = wid % WPB

    pltpu.sync_copy(sc_hbm.at[pl.ds(b * N, N)], sc_v)
    pltpu.sync_copy(meta_hbm.at[pl.ds(b * 32, 32)], meta_v)
    vvec = meta_v[pl.ds(0, 16)]
    needvec = meta_v[pl.ds(16, 16)]

    stop = (q + 1) * ROWS_PER_W

    def ccond(carry):
        i, off, eqrun = carry
        return jnp.logical_and(i < N // 16, off < stop)

    def cstep(carry):
        i, off, eqrun = carry
        sv = sc_v[pl.ds(i * 16, 16)]
        sbits = plsc.bitcast(sv, jnp.int32)
        gt = sbits > vvec
        eq = sbits == vvec
        eqc = plsc.cumsum(jnp.where(eq, 1, 0)) + eqrun
        sel = jnp.logical_or(gt, jnp.logical_and(eq, eqc <= needvec))
        idxv = lax.iota(jnp.int32, 16) + i * 16
        plsc.store_compressed(idx_v.at[pl.ds(off, 16)], idxv, mask=sel)
        scl = sv / (sv + EPS)
        plsc.store_compressed(scale_v.at[pl.ds(off, 16)], scl, mask=sel)
        npop = jnp.sum(jnp.where(sel, 1, 0))
        neq = jnp.sum(jnp.where(eq, 1, 0))
        return (i + 1, off + npop, eqrun + neq)

    lax.while_loop(ccond, cstep, (jnp.int32(0), jnp.int32(0), jnp.int32(0)))

    @pl.when(q == WPB - 1)
    def _():
        pltpu.sync_copy(idx_v.at[pl.ds(0, K)], tidx_hbm.at[b])
    pltpu.sync_copy(scale_v.at[pl.ds(q * ROWS_PER_W, ROWS_PER_W)],
                    scl_hbm.at[pl.ds(b * K + q * ROWS_PER_W, ROWS_PER_W)])

    NCH = ROWS_PER_W // CHUNK

    def fire(ch, buf):
        jb = q * ROWS_PER_W + ch * CHUNK
        for v4 in range(CHUNK // 16):
            gidx_v[pl.ds(buf * CHUNK + v4 * 16, 16)] = (
                idx_v[pl.ds(jb + v4 * 16, 16)] + b * N)
        return pltpu.async_copy(
            xt_hbm.at[gidx_v.at[pl.ds(buf * CHUNK, CHUNK)]],
            rows_v.at[buf], sem)

    cp = fire(0, 0)
    wcp = [None, None]
    for ch in range(NCH):
        buf = ch % 2
        cp.wait()
        if ch + 1 < NCH:
            if wcp[1 - buf] is not None:
                wcp[1 - buf].wait()
                wcp[1 - buf] = None
            cp = fire(ch + 1, 1 - buf)
        jb = q * ROWS_PER_W + ch * CHUNK
        wcp[buf] = pltpu.async_copy(
            rows_v.at[buf], out_hbm.at[pl.ds(b * K + jb, CHUNK)], wsem[buf])
    for w in wcp:
        if w is not None:
            w.wait()


@functools.cache
def _make_sc_gather():
    return functools.partial(
        pl.kernel,
        mesh=plsc.VectorSubcoreMesh(core_axis_name="c", subcore_axis_name="s"),
        out_type=[
            jax.ShapeDtypeStruct((B * K, D), jnp.float32),
            jax.ShapeDtypeStruct((B, K), jnp.int32),
            jax.ShapeDtypeStruct((B * K,), jnp.float32),
        ],
        scratch_types=[
            pltpu.VMEM((N,), jnp.float32),
            pltpu.VMEM((K + 16,), jnp.int32),
            pltpu.VMEM((K + 16,), jnp.float32),
            pltpu.VMEM((32,), jnp.int32),
            pltpu.VMEM((2 * CHUNK,), jnp.int32),
            pltpu.VMEM((2, CHUNK, D), jnp.float32),
            pltpu.SemaphoreType.DMA,
            (pltpu.SemaphoreType.DMA, pltpu.SemaphoreType.DMA),
        ],
        compiler_params=pltpu.CompilerParams(needs_layout_passes=False),
    )(_sc_body)


def _sc_gather(xt_flat, sc_flat, meta_flat):
    return _make_sc_gather()(xt_flat, sc_flat, meta_flat)


KB = 2048


def _ot_body(in_ref, s_ref, out_ref):
    vt = in_ref[...].T
    vt = vt * s_ref[...][None, :]
    out_ref[0] = vt.reshape(T, C, KB)


def _out_transpose(out_flat, scl):
    return pl.pallas_call(
        _ot_body,
        grid=(B, K // KB),
        in_specs=[pl.BlockSpec((KB, D), lambda b, k: (b * (K // KB) + k, 0)),
                  pl.BlockSpec((KB,), lambda b, k: (b * (K // KB) + k,))],
        out_specs=pl.BlockSpec((1, T, C, KB), lambda b, k: (b, 0, 0, k)),
        out_shape=jax.ShapeDtypeStruct((B, T, C, K), jnp.float32),
        compiler_params=pltpu.CompilerParams(
            dimension_semantics=("arbitrary", "arbitrary")),
    )(out_flat, scl)


def _router_conv(x, w, b):
    y = lax.conv_general_dilated(x, w, (1, 1), 'SAME',
                                 dimension_numbers=('NCHW', 'OIHW', 'NCHW'))
    return y + b[None, :, None, None]


def kernel(x, W1, b1, W2, b2):
    x5 = jnp.transpose(x, (0, 1, 3, 4, 2))
    xt, mean = _mean_transpose(x5)
    xm = mean.reshape(B, C, H, W)
    h = _router_conv(xm, W1, b1)
    h = jnp.where(h >= 0, h, 0.01 * h)
    s = _router_conv(h, W2, b2)
    scores = jax.nn.sigmoid(s).reshape(B, N)
    meta = _topk_meta(scores)
    out_flat, top_idx, scl = _sc_gather(xt.reshape(B * N, D),
                                        scores.reshape(B * N),
                                        meta.reshape(B * 32))
    out_tck = _out_transpose(out_flat, scl)
    return jnp.transpose(out_tck, (0, 3, 1, 2)), top_idx
